# Initial kernel scaffold; baseline (speedup 1.0000x reference)
import jax
import jax.numpy as jnp
from jax.experimental import pallas as pl


def kernel(h, t, r, input_y, ent_embeddings_1, ent_embeddings_2, rel_embeddings_1, rel_embeddings_2, ent_embeddings, rel_embeddings):
    raise NotImplementedError("write your pallas kernel here")


# speedup vs baseline: 1.4814x; 1.4814x over previous
import functools

import jax
import jax.numpy as jnp
from jax import lax
from jax.experimental import pallas as pl
from jax.experimental.pallas import tpu as pltpu
from jax.experimental.pallas import tpu_sc as plsc

B = 16384
L2_REG_LAMBDA = 0.001

NC = 2
NS = 16
NW = NC * NS
BPW = B // NW
CH = 128
NCHUNK = BPW // CH


def _sc_gather_score(h, t, r, e1, e2, r1, r2, ee, re):
    mesh = plsc.VectorSubcoreMesh(
        core_axis_name="c", subcore_axis_name="s", num_cores=NC, num_subcores=NS
    )

    @functools.partial(
        pl.kernel,
        mesh=mesh,
        compiler_params=pltpu.CompilerParams(use_tc_tiling_on_sc=False),
        out_type=(
            jax.ShapeDtypeStruct((B,), jnp.float32),
            jax.ShapeDtypeStruct((2, NW * 16), jnp.float32),
        ),
        scratch_types=(
            pltpu.VMEM((CH,), jnp.int32),
            pltpu.VMEM((CH,), jnp.int32),
            pltpu.VMEM((CH,), jnp.int32),
            pltpu.VMEM((CH, 32), jnp.float32),
            pltpu.VMEM((CH, 32), jnp.float32),
            pltpu.VMEM((CH, 32), jnp.float32),
            pltpu.VMEM((CH, 32), jnp.float32),
            pltpu.VMEM((CH, 32), jnp.float32),
            pltpu.VMEM((CH, 32), jnp.float32),
            pltpu.VMEM((CH, 64), jnp.float32),
            pltpu.VMEM((CH, 64), jnp.float32),
            pltpu.VMEM((CH, 64), jnp.float32),
            pltpu.VMEM((BPW,), jnp.float32),
            pltpu.VMEM((2, 16), jnp.float32),
            pltpu.SemaphoreType.DMA,
        ),
    )
    def k(h_hbm, t_hbm, r_hbm, e1_hbm, e2_hbm, r1_hbm, r2_hbm, ee_hbm, re_hbm,
          pred_hbm, acc_hbm,
          hi, ti, ri, e1h, e2h, e1t, e2t, r1g, r2g, ehg, etg, erg,
          predv, accv, sem):
        lane_iota = lax.iota(jnp.int32, 16)
        wid = lax.axis_index("s") * NC + lax.axis_index("c")
        base = pl.multiple_of(wid * BPW, BPW)
        acc32 = jnp.zeros((16,), jnp.float32)
        acc64 = jnp.zeros((16,), jnp.float32)
        for c in range(NCHUNK):
            off = base + c * CH
            pltpu.sync_copy(h_hbm.at[pl.ds(off, CH)], hi)
            pltpu.sync_copy(t_hbm.at[pl.ds(off, CH)], ti)
            pltpu.sync_copy(r_hbm.at[pl.ds(off, CH)], ri)
            cps = (
                pltpu.async_copy(e1_hbm.at[hi], e1h, sem),
                pltpu.async_copy(e2_hbm.at[hi], e2h, sem),
                pltpu.async_copy(e1_hbm.at[ti], e1t, sem),
                pltpu.async_copy(e2_hbm.at[ti], e2t, sem),
                pltpu.async_copy(r1_hbm.at[ri], r1g, sem),
                pltpu.async_copy(r2_hbm.at[ri], r2g, sem),
                pltpu.async_copy(ee_hbm.at[hi], ehg, sem),
                pltpu.async_copy(ee_hbm.at[ti], etg, sem),
                pltpu.async_copy(re_hbm.at[ri], erg, sem),
            )
            for cp in cps:
                cp.wait()

            def group(g, carry, c=c):
                a32, a64 = carry
                g16 = pl.multiple_of(g * 16, 16)
                tot = jnp.zeros((16,), jnp.float32)
                for j in range(16):
                    i = g16 + j
                    s = None
                    for half in range(2):
                        sl = pl.ds(half * 16, 16)
                        a1 = e1h[i, sl]
                        a2 = e2h[i, sl]
                        b1 = e1t[i, sl]
                        b2 = e2t[i, sl]
                        q1 = r1g[i, sl]
                        q2 = r2g[i, sl]
                        contrib = ((a1 * b1 + a2 * b2) * q1
                                   + (a1 * b2 - a2 * b1) * q2)
                        s = contrib if s is None else s + contrib
                        a32 = (a32 + a1 * a1 + a2 * a2 + b1 * b1 + b2 * b2
                               + q1 * q1 + q2 * q2)
                    for q in range(4):
                        sl = pl.ds(q * 16, 16)
                        hh = ehg[i, sl]
                        tt = etg[i, sl]
                        rr = erg[i, sl]
                        s = s + hh * tt * rr
                        a64 = a64 + hh * hh + tt * tt + rr * rr
                    p0 = (s[0] + s[1]) + (s[2] + s[3])
                    p1 = (s[4] + s[5]) + (s[6] + s[7])
                    p2 = (s[8] + s[9]) + (s[10] + s[11])
                    p3 = (s[12] + s[13]) + (s[14] + s[15])
                    rsum = (p0 + p1) + (p2 + p3)
                    tot = jnp.where(lane_iota == j, rsum, tot)
                predv[pl.ds(c * CH + g16, 16)] = -tot
                return a32, a64

            acc32, acc64 = lax.fori_loop(0, CH // 16, group, (acc32, acc64))
        accv[0, :] = acc32
        accv[1, :] = acc64
        pltpu.sync_copy(predv, pred_hbm.at[pl.ds(base, BPW)])
        lane = pl.multiple_of(wid * 16, 16)
        pltpu.sync_copy(accv.at[0], acc_hbm.at[0, pl.ds(lane, 16)])
        pltpu.sync_copy(accv.at[1], acc_hbm.at[1, pl.ds(lane, 16)])

    return k(h, t, r, e1, e2, r1, r2, ee, re)


def _tc_loss(pred, y, accs):
    def body(p_ref, y_ref, a_ref, o_ref):
        z = y_ref[...] * p_ref[...]
        sp = jnp.maximum(z, 0.0) + jnp.log1p(jnp.exp(-jnp.abs(z)))
        loss_f = jnp.sum(sp) * (1.0 / B)
        s32 = jnp.sum(a_ref[0:1, :])
        s64 = jnp.sum(a_ref[1:2, :])
        o_ref[0, 0] = loss_f + L2_REG_LAMBDA * (
            s32 * (1.0 / (B * 32.0)) + s64 * (1.0 / (B * 64.0))
        )

    out = pl.pallas_call(
        body,
        out_shape=jax.ShapeDtypeStruct((1, 1), jnp.float32),
        out_specs=pl.BlockSpec(memory_space=pltpu.SMEM),
    )(pred.reshape(128, 128), y.reshape(128, 128), accs)
    return out.reshape(())


def kernel(h, t, r, input_y, ent_embeddings_1, ent_embeddings_2,
           rel_embeddings_1, rel_embeddings_2, ent_embeddings, rel_embeddings):
    pred, accs = _sc_gather_score(
        h, t, r, ent_embeddings_1, ent_embeddings_2,
        rel_embeddings_1, rel_embeddings_2, ent_embeddings, rel_embeddings)
    return _tc_loss(pred, input_y, accs)


# speedup vs baseline: 1.5579x; 1.0516x over previous
import functools

import jax
import jax.numpy as jnp
from jax import lax
from jax.experimental import pallas as pl
from jax.experimental.pallas import tpu as pltpu
from jax.experimental.pallas import tpu_sc as plsc

B = 16384
L2_REG_LAMBDA = 0.001

NC = 2
NS = 16
NW = NC * NS
BPW = B // NW
CH = 128
NCHUNK = BPW // CH


def _sc_gather_score(h, t, r, e1, e2, r1, r2, ee, re):
    mesh = plsc.VectorSubcoreMesh(
        core_axis_name="c", subcore_axis_name="s", num_cores=NC, num_subcores=NS
    )

    gather_set = (
        pltpu.VMEM((CH, 32), jnp.float32),
        pltpu.VMEM((CH, 32), jnp.float32),
        pltpu.VMEM((CH, 32), jnp.float32),
        pltpu.VMEM((CH, 32), jnp.float32),
        pltpu.VMEM((CH, 32), jnp.float32),
        pltpu.VMEM((CH, 32), jnp.float32),
        pltpu.VMEM((CH, 64), jnp.float32),
        pltpu.VMEM((CH, 64), jnp.float32),
        pltpu.VMEM((CH, 64), jnp.float32),
    )

    @functools.partial(
        pl.kernel,
        mesh=mesh,
        compiler_params=pltpu.CompilerParams(use_tc_tiling_on_sc=False),
        out_type=(
            jax.ShapeDtypeStruct((B,), jnp.float32),
            jax.ShapeDtypeStruct((2, NW * 16), jnp.float32),
        ),
        scratch_types=(
            pltpu.VMEM((BPW,), jnp.int32),
            pltpu.VMEM((BPW,), jnp.int32),
            pltpu.VMEM((BPW,), jnp.int32),
            gather_set,
            gather_set,
            pltpu.VMEM((BPW,), jnp.float32),
            pltpu.VMEM((2, 16), jnp.float32),
            pltpu.SemaphoreType.DMA,
            pltpu.SemaphoreType.DMA,
        ),
    )
    def k(h_hbm, t_hbm, r_hbm, e1_hbm, e2_hbm, r1_hbm, r2_hbm, ee_hbm, re_hbm,
          pred_hbm, acc_hbm,
          hi, ti, ri, bufs_a, bufs_b, predv, accv, sem_a, sem_b):
        lane_iota = lax.iota(jnp.int32, 16)
        wid = lax.axis_index("s") * NC + lax.axis_index("c")
        base = pl.multiple_of(wid * BPW, BPW)
        pltpu.sync_copy(h_hbm.at[pl.ds(base, BPW)], hi)
        pltpu.sync_copy(t_hbm.at[pl.ds(base, BPW)], ti)
        pltpu.sync_copy(r_hbm.at[pl.ds(base, BPW)], ri)

        bufs = (bufs_a, bufs_b)
        sems = (sem_a, sem_b)

        def fire(c):
            e1h, e2h, e1t, e2t, r1g, r2g, ehg, etg, erg = bufs[c % 2]
            sem = sems[c % 2]
            hs = hi.at[pl.ds(c * CH, CH)]
            ts = ti.at[pl.ds(c * CH, CH)]
            rs = ri.at[pl.ds(c * CH, CH)]
            return (
                pltpu.async_copy(e1_hbm.at[hs], e1h, sem),
                pltpu.async_copy(e2_hbm.at[hs], e2h, sem),
                pltpu.async_copy(e1_hbm.at[ts], e1t, sem),
                pltpu.async_copy(e2_hbm.at[ts], e2t, sem),
                pltpu.async_copy(r1_hbm.at[rs], r1g, sem),
                pltpu.async_copy(r2_hbm.at[rs], r2g, sem),
                pltpu.async_copy(ee_hbm.at[hs], ehg, sem),
                pltpu.async_copy(ee_hbm.at[ts], etg, sem),
                pltpu.async_copy(re_hbm.at[rs], erg, sem),
            )

        acc32 = jnp.zeros((16,), jnp.float32)
        acc64 = jnp.zeros((16,), jnp.float32)
        cps = fire(0)
        for c in range(NCHUNK):
            nxt = fire(c + 1) if c + 1 < NCHUNK else ()
            for cp in cps:
                cp.wait()
            cps = nxt
            e1h, e2h, e1t, e2t, r1g, r2g, ehg, etg, erg = bufs[c % 2]

            def group(g, carry, c=c, e1h=e1h, e2h=e2h, e1t=e1t, e2t=e2t,
                      r1g=r1g, r2g=r2g, ehg=ehg, etg=etg, erg=erg):
                a32, a64 = carry
                g16 = pl.multiple_of(g * 16, 16)
                tot = jnp.zeros((16,), jnp.float32)
                for j in range(16):
                    i = g16 + j
                    s = None
                    for half in range(2):
                        sl = pl.ds(half * 16, 16)
                        a1 = e1h[i, sl]
                        a2 = e2h[i, sl]
                        b1 = e1t[i, sl]
                        b2 = e2t[i, sl]
                        q1 = r1g[i, sl]
                        q2 = r2g[i, sl]
                        contrib = ((a1 * b1 + a2 * b2) * q1
                                   + (a1 * b2 - a2 * b1) * q2)
                        s = contrib if s is None else s + contrib
                        a32 = (a32 + a1 * a1 + a2 * a2 + b1 * b1 + b2 * b2
                               + q1 * q1 + q2 * q2)
                    for q in range(4):
                        sl = pl.ds(q * 16, 16)
                        hh = ehg[i, sl]
                        tt = etg[i, sl]
                        rr = erg[i, sl]
                        s = s + hh * tt * rr
                        a64 = a64 + hh * hh + tt * tt + rr * rr
                    p0 = (s[0] + s[1]) + (s[2] + s[3])
                    p1 = (s[4] + s[5]) + (s[6] + s[7])
                    p2 = (s[8] + s[9]) + (s[10] + s[11])
                    p3 = (s[12] + s[13]) + (s[14] + s[15])
                    rsum = (p0 + p1) + (p2 + p3)
                    tot = jnp.where(lane_iota == j, rsum, tot)
                predv[pl.ds(c * CH + g16, 16)] = -tot
                return a32, a64

            acc32, acc64 = lax.fori_loop(0, CH // 16, group, (acc32, acc64))
        accv[0, :] = acc32
        accv[1, :] = acc64
        pltpu.sync_copy(predv, pred_hbm.at[pl.ds(base, BPW)])
        lane = pl.multiple_of(wid * 16, 16)
        pltpu.sync_copy(accv.at[0], acc_hbm.at[0, pl.ds(lane, 16)])
        pltpu.sync_copy(accv.at[1], acc_hbm.at[1, pl.ds(lane, 16)])

    return k(h, t, r, e1, e2, r1, r2, ee, re)


def _tc_loss(pred, y, accs):
    def body(p_ref, y_ref, a_ref, o_ref):
        z = y_ref[...] * p_ref[...]
        sp = jnp.maximum(z, 0.0) + jnp.log1p(jnp.exp(-jnp.abs(z)))
        loss_f = jnp.sum(sp) * (1.0 / B)
        s32 = jnp.sum(a_ref[0:1, :])
        s64 = jnp.sum(a_ref[1:2, :])
        o_ref[0, 0] = loss_f + L2_REG_LAMBDA * (
            s32 * (1.0 / (B * 32.0)) + s64 * (1.0 / (B * 64.0))
        )

    out = pl.pallas_call(
        body,
        out_shape=jax.ShapeDtypeStruct((1, 1), jnp.float32),
        out_specs=pl.BlockSpec(memory_space=pltpu.SMEM),
    )(pred.reshape(128, 128), y.reshape(128, 128), accs)
    return out.reshape(())


def kernel(h, t, r, input_y, ent_embeddings_1, ent_embeddings_2,
           rel_embeddings_1, rel_embeddings_2, ent_embeddings, rel_embeddings):
    pred, accs = _sc_gather_score(
        h, t, r, ent_embeddings_1, ent_embeddings_2,
        rel_embeddings_1, rel_embeddings_2, ent_embeddings, rel_embeddings)
    return _tc_loss(pred, input_y, accs)


# speedup vs baseline: 1.5582x; 1.0001x over previous
import functools

import jax
import jax.numpy as jnp
from jax import lax
from jax.experimental import pallas as pl
from jax.experimental.pallas import tpu as pltpu
from jax.experimental.pallas import tpu_sc as plsc

B = 16384
L2_REG_LAMBDA = 0.001

NC = 2
NS = 16
NW = NC * NS
BPW = B // NW
CH = 128
NCHUNK = BPW // CH


def _sc_gather_score(h, t, r, e1, e2, r1, r2, ee, re):
    mesh = plsc.VectorSubcoreMesh(
        core_axis_name="c", subcore_axis_name="s", num_cores=NC, num_subcores=NS
    )

    gather_set = (
        pltpu.VMEM((CH, 32), jnp.float32),
        pltpu.VMEM((CH, 32), jnp.float32),
        pltpu.VMEM((CH, 32), jnp.float32),
        pltpu.VMEM((CH, 32), jnp.float32),
        pltpu.VMEM((CH, 32), jnp.float32),
        pltpu.VMEM((CH, 32), jnp.float32),
        pltpu.VMEM((CH, 64), jnp.float32),
        pltpu.VMEM((CH, 64), jnp.float32),
        pltpu.VMEM((CH, 64), jnp.float32),
    )

    @functools.partial(
        pl.kernel,
        mesh=mesh,
        compiler_params=pltpu.CompilerParams(use_tc_tiling_on_sc=False),
        out_type=(
            jax.ShapeDtypeStruct((B,), jnp.float32),
            jax.ShapeDtypeStruct((2, NW * 16), jnp.float32),
        ),
        scratch_types=(
            pltpu.VMEM((BPW,), jnp.int32),
            pltpu.VMEM((BPW,), jnp.int32),
            pltpu.VMEM((BPW,), jnp.int32),
            gather_set,
            gather_set,
            pltpu.VMEM((BPW,), jnp.float32),
            pltpu.VMEM((2, 16), jnp.float32),
            pltpu.SemaphoreType.DMA,
            pltpu.SemaphoreType.DMA,
        ),
    )
    def k(ee_hbm, h_hbm, t_hbm, r_hbm, e1_hbm, e2_hbm, r1_hbm, r2_hbm, re_hbm,
          pred_hbm, acc_hbm,
          hi, ti, ri, bufs_a, bufs_b, predv, accv, sem_a, sem_b):
        lane_iota = lax.iota(jnp.int32, 16)
        wid = lax.axis_index("s") * NC + lax.axis_index("c")
        base = pl.multiple_of(wid * BPW, BPW)
        pltpu.sync_copy(h_hbm.at[pl.ds(base, BPW)], hi)
        pltpu.sync_copy(t_hbm.at[pl.ds(base, BPW)], ti)
        pltpu.sync_copy(r_hbm.at[pl.ds(base, BPW)], ri)

        bufs = (bufs_a, bufs_b)
        sems = (sem_a, sem_b)

        def fire(c):
            e1h, e2h, e1t, e2t, r1g, r2g, ehg, etg, erg = bufs[c % 2]
            sem = sems[c % 2]
            hs = hi.at[pl.ds(c * CH, CH)]
            ts = ti.at[pl.ds(c * CH, CH)]
            rs = ri.at[pl.ds(c * CH, CH)]
            return (
                pltpu.async_copy(e1_hbm.at[hs], e1h, sem),
                pltpu.async_copy(e2_hbm.at[hs], e2h, sem),
                pltpu.async_copy(e1_hbm.at[ts], e1t, sem),
                pltpu.async_copy(e2_hbm.at[ts], e2t, sem),
                pltpu.async_copy(r1_hbm.at[rs], r1g, sem),
                pltpu.async_copy(r2_hbm.at[rs], r2g, sem),
                pltpu.async_copy(ee_hbm.at[hs], ehg, sem),
                pltpu.async_copy(ee_hbm.at[ts], etg, sem),
                pltpu.async_copy(re_hbm.at[rs], erg, sem),
            )

        acc32 = jnp.zeros((16,), jnp.float32)
        acc64 = jnp.zeros((16,), jnp.float32)
        cps = fire(0)
        for c in range(NCHUNK):
            nxt = fire(c + 1) if c + 1 < NCHUNK else ()
            for cp in cps:
                cp.wait()
            cps = nxt
            e1h, e2h, e1t, e2t, r1g, r2g, ehg, etg, erg = bufs[c % 2]

            def group(g, carry, c=c, e1h=e1h, e2h=e2h, e1t=e1t, e2t=e2t,
                      r1g=r1g, r2g=r2g, ehg=ehg, etg=etg, erg=erg):
                a32, a64 = carry
                g16 = pl.multiple_of(g * 16, 16)
                tot = jnp.zeros((16,), jnp.float32)
                for j in range(16):
                    i = g16 + j
                    s = None
                    for half in range(2):
                        sl = pl.ds(half * 16, 16)
                        a1 = e1h[i, sl]
                        a2 = e2h[i, sl]
                        b1 = e1t[i, sl]
                        b2 = e2t[i, sl]
                        q1 = r1g[i, sl]
                        q2 = r2g[i, sl]
                        contrib = ((a1 * b1 + a2 * b2) * q1
                                   + (a1 * b2 - a2 * b1) * q2)
                        s = contrib if s is None else s + contrib
                        a32 = (a32 + a1 * a1 + a2 * a2 + b1 * b1 + b2 * b2
                               + q1 * q1 + q2 * q2)
                    for q in range(4):
                        sl = pl.ds(q * 16, 16)
                        hh = ehg[i, sl]
                        tt = etg[i, sl]
                        rr = erg[i, sl]
                        s = s + hh * tt * rr
                        a64 = a64 + hh * hh + tt * tt + rr * rr
                    p0 = (s[0] + s[1]) + (s[2] + s[3])
                    p1 = (s[4] + s[5]) + (s[6] + s[7])
                    p2 = (s[8] + s[9]) + (s[10] + s[11])
                    p3 = (s[12] + s[13]) + (s[14] + s[15])
                    rsum = (p0 + p1) + (p2 + p3)
                    tot = jnp.where(lane_iota == j, rsum, tot)
                predv[pl.ds(c * CH + g16, 16)] = -tot
                return a32, a64

            acc32, acc64 = lax.fori_loop(0, CH // 16, group, (acc32, acc64))
        accv[0, :] = acc32
        accv[1, :] = acc64
        pltpu.sync_copy(predv, pred_hbm.at[pl.ds(base, BPW)])
        lane = pl.multiple_of(wid * 16, 16)
        pltpu.sync_copy(accv.at[0], acc_hbm.at[0, pl.ds(lane, 16)])
        pltpu.sync_copy(accv.at[1], acc_hbm.at[1, pl.ds(lane, 16)])

    return k(ee, h, t, r, e1, e2, r1, r2, re)


def _tc_loss(pred, y, accs):
    def body(p_ref, y_ref, a_ref, o_ref):
        z = y_ref[...] * p_ref[...]
        sp = jnp.maximum(z, 0.0) + jnp.log1p(jnp.exp(-jnp.abs(z)))
        loss_f = jnp.sum(sp) * (1.0 / B)
        s32 = jnp.sum(a_ref[0:1, :])
        s64 = jnp.sum(a_ref[1:2, :])
        o_ref[0, 0] = loss_f + L2_REG_LAMBDA * (
            s32 * (1.0 / (B * 32.0)) + s64 * (1.0 / (B * 64.0))
        )

    out = pl.pallas_call(
        body,
        out_shape=jax.ShapeDtypeStruct((1, 1), jnp.float32),
        out_specs=pl.BlockSpec(memory_space=pltpu.SMEM),
    )(pred.reshape(128, 128), y.reshape(128, 128), accs)
    return out.reshape(())


def kernel(h, t, r, input_y, ent_embeddings_1, ent_embeddings_2,
           rel_embeddings_1, rel_embeddings_2, ent_embeddings, rel_embeddings):
    pred, accs = _sc_gather_score(
        h, t, r, ent_embeddings_1, ent_embeddings_2,
        rel_embeddings_1, rel_embeddings_2, ent_embeddings, rel_embeddings)
    return _tc_loss(pred, input_y, accs)


# speedup vs baseline: 1.5849x; 1.0172x over previous
import functools

import jax
import jax.numpy as jnp
from jax import lax
from jax.experimental import pallas as pl
from jax.experimental.pallas import tpu as pltpu
from jax.experimental.pallas import tpu_sc as plsc

B = 16384
L2_REG_LAMBDA = 0.001

NC = 2
NS = 16
NW = NC * NS
BPW = B // NW
CH = 128
NCHUNK = BPW // CH


def _sc_gather_score(h, t, r, e1, e2, r1, r2, ee, re):
    mesh = plsc.VectorSubcoreMesh(
        core_axis_name="c", subcore_axis_name="s", num_cores=NC, num_subcores=NS
    )

    gather_set = (
        pltpu.VMEM((CH, 32), jnp.float32),
        pltpu.VMEM((CH, 32), jnp.float32),
        pltpu.VMEM((CH, 32), jnp.float32),
        pltpu.VMEM((CH, 32), jnp.float32),
        pltpu.VMEM((CH, 32), jnp.float32),
        pltpu.VMEM((CH, 32), jnp.float32),
        pltpu.VMEM((CH, 64), jnp.float32),
        pltpu.VMEM((CH, 64), jnp.float32),
        pltpu.VMEM((CH, 64), jnp.float32),
    )

    @functools.partial(
        pl.kernel,
        mesh=mesh,
        compiler_params=pltpu.CompilerParams(use_tc_tiling_on_sc=False),
        out_type=(
            jax.ShapeDtypeStruct((B, 16), jnp.float32),
            jax.ShapeDtypeStruct((2, NW * 16), jnp.float32),
        ),
        scratch_types=(
            pltpu.VMEM((BPW,), jnp.int32),
            pltpu.VMEM((BPW,), jnp.int32),
            pltpu.VMEM((BPW,), jnp.int32),
            gather_set,
            gather_set,
            pltpu.VMEM((CH, 16), jnp.float32),
            pltpu.VMEM((CH, 16), jnp.float32),
            pltpu.VMEM((2, 16), jnp.float32),
            pltpu.SemaphoreType.DMA,
            pltpu.SemaphoreType.DMA,
            pltpu.SemaphoreType.DMA,
        ),
    )
    def k(ee_hbm, h_hbm, t_hbm, r_hbm, e1_hbm, e2_hbm, r1_hbm, r2_hbm, re_hbm,
          pacc_hbm, acc_hbm,
          hi, ti, ri, bufs_a, bufs_b, pacc_a, pacc_b, accv,
          sem_a, sem_b, sem_o):
        wid = lax.axis_index("s") * NC + lax.axis_index("c")
        base = pl.multiple_of(wid * BPW, BPW)
        pltpu.sync_copy(h_hbm.at[pl.ds(base, BPW)], hi)
        pltpu.sync_copy(t_hbm.at[pl.ds(base, BPW)], ti)
        pltpu.sync_copy(r_hbm.at[pl.ds(base, BPW)], ri)

        bufs = (bufs_a, bufs_b)
        paccs = (pacc_a, pacc_b)
        sems = (sem_a, sem_b)

        def fire(c):
            e1h, e2h, e1t, e2t, r1g, r2g, ehg, etg, erg = bufs[c % 2]
            sem = sems[c % 2]
            hs = hi.at[pl.ds(c * CH, CH)]
            ts = ti.at[pl.ds(c * CH, CH)]
            rs = ri.at[pl.ds(c * CH, CH)]
            return (
                pltpu.async_copy(ee_hbm.at[hs], ehg, sem),
                pltpu.async_copy(ee_hbm.at[ts], etg, sem),
                pltpu.async_copy(e1_hbm.at[hs], e1h, sem),
                pltpu.async_copy(e2_hbm.at[hs], e2h, sem),
                pltpu.async_copy(e1_hbm.at[ts], e1t, sem),
                pltpu.async_copy(e2_hbm.at[ts], e2t, sem),
                pltpu.async_copy(r1_hbm.at[rs], r1g, sem),
                pltpu.async_copy(r2_hbm.at[rs], r2g, sem),
                pltpu.async_copy(re_hbm.at[rs], erg, sem),
            )

        acc32 = jnp.zeros((16,), jnp.float32)
        acc64 = jnp.zeros((16,), jnp.float32)
        cps = fire(0)
        out_cp = None
        for c in range(NCHUNK):
            nxt = fire(c + 1) if c + 1 < NCHUNK else ()
            for cp in cps:
                cp.wait()
            cps = nxt
            e1h, e2h, e1t, e2t, r1g, r2g, ehg, etg, erg = bufs[c % 2]
            pacc = paccs[c % 2]

            def row(i, carry, e1h=e1h, e2h=e2h, e1t=e1t, e2t=e2t,
                    r1g=r1g, r2g=r2g, ehg=ehg, etg=etg, erg=erg, pacc=pacc):
                a32, a64 = carry
                s = None
                for half in range(2):
                    sl = pl.ds(half * 16, 16)
                    a1 = e1h[i, sl]
                    a2 = e2h[i, sl]
                    b1 = e1t[i, sl]
                    b2 = e2t[i, sl]
                    q1 = r1g[i, sl]
                    q2 = r2g[i, sl]
                    contrib = ((a1 * b1 + a2 * b2) * q1
                               + (a1 * b2 - a2 * b1) * q2)
                    s = contrib if s is None else s + contrib
                    sq = (((a1 * a1 + a2 * a2) + (b1 * b1 + b2 * b2))
                          + (q1 * q1 + q2 * q2))
                    a32 = a32 + sq
                d64 = None
                for q in range(4):
                    sl = pl.ds(q * 16, 16)
                    hh = ehg[i, sl]
                    tt = etg[i, sl]
                    rr = erg[i, sl]
                    s = s + hh * tt * rr
                    sq = (hh * hh + tt * tt) + rr * rr
                    d64 = sq if d64 is None else d64 + sq
                a64 = a64 + d64
                pacc[i, :] = s
                return a32, a64

            acc32, acc64 = lax.fori_loop(0, CH, row, (acc32, acc64))
            if out_cp is not None:
                out_cp.wait()
            out_cp = pltpu.async_copy(
                pacc, pacc_hbm.at[pl.ds(base + c * CH, CH)], sem_o)
        out_cp.wait()
        accv[0, :] = acc32
        accv[1, :] = acc64
        lane = pl.multiple_of(wid * 16, 16)
        pltpu.sync_copy(accv.at[0], acc_hbm.at[0, pl.ds(lane, 16)])
        pltpu.sync_copy(accv.at[1], acc_hbm.at[1, pl.ds(lane, 16)])

    return k(ee, h, t, r, e1, e2, r1, r2, re)


def _tc_loss(pacc, y, accs):
    def body(p_ref, y_ref, a_ref, o_ref):
        x = p_ref[...]
        rows = lax.broadcasted_iota(jnp.int32, (2048, 128), 0)
        cols = lax.broadcasted_iota(jnp.int32, (2048, 128), 1)
        m = jnp.where(rows // 16 == cols, 1.0, 0.0).astype(jnp.float32)
        s = jax.lax.dot_general(
            x, m, (((1,), (0,)), ((), ())),
            preferred_element_type=jnp.float32)
        z = y_ref[...] * (-s)
        sp = jnp.maximum(z, 0.0) + jnp.log1p(jnp.exp(-jnp.abs(z)))
        loss_f = jnp.sum(sp) * (1.0 / B)
        s32 = jnp.sum(a_ref[0:1, :])
        s64 = jnp.sum(a_ref[1:2, :])
        o_ref[0, 0] = loss_f + L2_REG_LAMBDA * (
            s32 * (1.0 / (B * 32.0)) + s64 * (1.0 / (B * 64.0))
        )

    out = pl.pallas_call(
        body,
        out_shape=jax.ShapeDtypeStruct((1, 1), jnp.float32),
        out_specs=pl.BlockSpec(memory_space=pltpu.SMEM),
    )(pacc.reshape(128, 2048), y.reshape(128, 128), accs)
    return out.reshape(())


def kernel(h, t, r, input_y, ent_embeddings_1, ent_embeddings_2,
           rel_embeddings_1, rel_embeddings_2, ent_embeddings, rel_embeddings):
    pacc, accs = _sc_gather_score(
        h, t, r, ent_embeddings_1, ent_embeddings_2,
        rel_embeddings_1, rel_embeddings_2, ent_embeddings, rel_embeddings)
    return _tc_loss(pacc, input_y, accs)
